# baseline (device time: 26511 ns/iter reference)
import jax
import jax.numpy as jnp
from jax import lax
from jax.experimental import pallas as pl
from jax.experimental.pallas import tpu as pltpu


def kernel(x):
    _, m, n = x.shape
    S = m // 2
    H = m // 4
    Q = m // 8
    E = m // 16

    def body(x_ref, w, xv, rbuf1, rbuf2, ssem, rsem, lsem):
        my = lax.axis_index("i")
        b0 = my & 1
        b1 = my >> 1
        gray = b0 ^ b1

        p1a, p2a = my ^ 1, my ^ 3
        fa, ga = gray, b1
        p1b, p2b = my ^ 3, my ^ 1
        fb, gb = b1, b0

        ha = fa * H
        qa = ha + ga * Q
        oqa = ha + (1 - ga) * Q
        sa = (1 - fa) * H
        hb = S + fb * H
        qb = hb + gb * Q
        oqb = hb + (1 - gb) * Q
        sb = S + (1 - fb) * H

        pa_first = (1 - ga) * Q
        pb_first = gb * Q

        pa_first_ = (1 - ga) * Q
        pb_first_ = gb * Q
        stage_regions = [
            (sa + pa_first_, 2 * E),
            (sb + pb_first_, 2 * E),
            (sa + Q - pa_first_, 2 * E),
            (sb + Q - pb_first_, 2 * E),
            (ha, H),
            (hb, H),
        ]
        lcs = []
        for i, (off, rows) in enumerate(stage_regions):
            c = pltpu.make_async_copy(
                x_ref.at[0, pl.ds(off, rows), :],
                xv.at[pl.ds(off, rows), :],
                lsem.at[i],
            )
            c.start()
            lcs.append(c)

        barrier_sem = pltpu.get_barrier_semaphore()
        for nbr in (p1a, p1b):
            pl.semaphore_signal(
                barrier_sem, inc=1,
                device_id=(nbr,), device_id_type=pl.DeviceIdType.MESH,
            )
        pl.semaphore_wait(barrier_sem, 2)

        def copy(src, dst, k, dev):
            return pltpu.make_async_remote_copy(
                src_ref=src, dst_ref=dst,
                send_sem=ssem.at[k], recv_sem=rsem.at[k],
                device_id=(dev,), device_id_type=pl.DeviceIdType.MESH,
            )

        def cast(off, rows):
            w[pl.ds(off, rows), :] = xv[pl.ds(off, rows), :].astype(
                jnp.bfloat16
            )

        a_rel = [pa_first, pa_first + E, Q - pa_first, Q - pa_first + E]
        b_rel = [pb_first, pb_first + E, Q - pb_first, Q - pb_first + E]
        stage_wait = {0: [0, 1], 2: [2, 3]}
        rs1a, rs1b = [], []
        for j in range(4):
            for i in stage_wait.get(j, []):
                lcs[i].wait()
            cast(sa + a_rel[j], E)
            r = copy(
                w.at[pl.ds(sa + a_rel[j], E)],
                rbuf1.at[0, pl.ds(a_rel[j], E)], j, p1a,
            )
            r.start()
            rs1a.append(r)
            cast(sb + b_rel[j], E)
            r = copy(
                w.at[pl.ds(sb + b_rel[j], E)],
                rbuf1.at[1, pl.ds(b_rel[j], E)], 4 + j, p1b,
            )
            r.start()
            rs1b.append(r)
        lcs[4].wait()
        cast(ha, H)
        lcs[5].wait()
        cast(hb, H)

        oq_rel = [(1 - ga) * Q, (1 - gb) * Q]
        q_rel = [ga * Q, gb * Q]
        rs2a, rs2b = [], []
        for j in range(2):
            rs1a[j].wait()
            w[pl.ds(oqa + j * E, E), :] = (
                w[pl.ds(oqa + j * E, E), :]
                + rbuf1[0, pl.ds(oq_rel[0] + j * E, E), :]
            )
            r = copy(w.at[pl.ds(oqa + j * E, E)], rbuf2.at[0, pl.ds(j * E, E)],
                     8 + j, p2a)
            r.start()
            rs2a.append(r)
            rs1b[j].wait()
            w[pl.ds(oqb + j * E, E), :] = (
                w[pl.ds(oqb + j * E, E), :]
                + rbuf1[1, pl.ds(oq_rel[1] + j * E, E), :]
            )
            r = copy(w.at[pl.ds(oqb + j * E, E)], rbuf2.at[1, pl.ds(j * E, E)],
                     10 + j, p2b)
            r.start()
            rs2b.append(r)
        for j in range(2):
            rs1a[2 + j].wait()
            w[pl.ds(qa + j * E, E), :] = (
                w[pl.ds(qa + j * E, E), :]
                + rbuf1[0, pl.ds(q_rel[0] + j * E, E), :]
            )
            rs1b[2 + j].wait()
            w[pl.ds(qb + j * E, E), :] = (
                w[pl.ds(qb + j * E, E), :]
                + rbuf1[1, pl.ds(q_rel[1] + j * E, E), :]
            )

        ag1a, ag1b, ag2 = [], [], []
        for j in range(2):
            rs2a[j].wait()
            w[pl.ds(qa + j * E, E), :] = (
                w[pl.ds(qa + j * E, E), :] + rbuf2[0, pl.ds(j * E, E), :]
            )
            r = copy(w.at[pl.ds(qa + j * E, E)], w.at[pl.ds(qa + j * E, E)],
                     12 + j, p2a)
            r.start()
            ag1a.append(r)
            r = copy(w.at[pl.ds(qa + j * E, E)], w.at[pl.ds(qa + j * E, E)],
                     16 + j, p1a)
            r.start()
            ag2.append(r)
            rs2b[j].wait()
            w[pl.ds(qb + j * E, E), :] = (
                w[pl.ds(qb + j * E, E), :] + rbuf2[1, pl.ds(j * E, E), :]
            )
            r = copy(w.at[pl.ds(qb + j * E, E)], w.at[pl.ds(qb + j * E, E)],
                     14 + j, p2b)
            r.start()
            ag1b.append(r)
            r = copy(w.at[pl.ds(qb + j * E, E)], w.at[pl.ds(qb + j * E, E)],
                     20 + j, p1b)
            r.start()
            ag2.append(r)

        for j in range(2):
            ag1a[j].wait()
            r = copy(w.at[pl.ds(oqa + j * E, E)], w.at[pl.ds(oqa + j * E, E)],
                     18 + j, p1a)
            r.start()
            ag2.append(r)
            ag1b[j].wait()
            r = copy(w.at[pl.ds(oqb + j * E, E)], w.at[pl.ds(oqb + j * E, E)],
                     22 + j, p1b)
            r.start()
            ag2.append(r)

        for r in ag2:
            r.wait()

    return pl.pallas_call(
        body,
        out_shape=jax.ShapeDtypeStruct((m, n), jnp.bfloat16),
        in_specs=[pl.BlockSpec(memory_space=pltpu.HBM)],
        out_specs=pl.BlockSpec(memory_space=pltpu.VMEM),
        scratch_shapes=[
            pltpu.VMEM((m, n), jnp.float32),
            pltpu.VMEM((2, H, n), jnp.bfloat16),
            pltpu.VMEM((2, Q, n), jnp.bfloat16),
            pltpu.SemaphoreType.DMA((24,)),
            pltpu.SemaphoreType.DMA((24,)),
            pltpu.SemaphoreType.DMA((6,)),
        ],
        compiler_params=pltpu.CompilerParams(collective_id=0),
    )(x)


# device time: 26135 ns/iter; 1.0144x vs baseline; 1.0144x over previous
import jax
import jax.numpy as jnp
from jax import lax
from jax.experimental import pallas as pl
from jax.experimental.pallas import tpu as pltpu

C = 2


def kernel(x):
    _, m, n = x.shape
    S = m // 2
    H = m // 4
    Q = m // 8
    E = Q // C
    NS = 12 * C

    def body(x_ref, w, rbuf1, rbuf2, ssem, rsem):
        my = lax.axis_index("i")
        b0 = my & 1
        b1 = my >> 1
        gray = b0 ^ b1

        p1a, p2a = my ^ 1, my ^ 3
        fa, ga = gray, b1
        p1b, p2b = my ^ 3, my ^ 1
        fb, gb = b1, b0

        ha = fa * H
        qa = ha + ga * Q
        oqa = ha + (1 - ga) * Q
        sa = (1 - fa) * H
        hb = S + fb * H
        qb = hb + gb * Q
        oqb = hb + (1 - gb) * Q
        sb = S + (1 - fb) * H

        pa_first = (1 - ga) * Q
        pb_first = gb * Q

        barrier_sem = pltpu.get_barrier_semaphore()
        for nbr in (p1a, p1b):
            pl.semaphore_signal(
                barrier_sem, inc=1,
                device_id=(nbr,), device_id_type=pl.DeviceIdType.MESH,
            )
        pl.semaphore_wait(barrier_sem, 2)

        def copy(src, dst, k, dev):
            return pltpu.make_async_remote_copy(
                src_ref=src, dst_ref=dst,
                send_sem=ssem.at[k], recv_sem=rsem.at[k],
                device_id=(dev,), device_id_type=pl.DeviceIdType.MESH,
            )

        def cast(off, rows):
            w[pl.ds(off, rows), :] = x_ref[0, pl.ds(off, rows), :].astype(
                jnp.bfloat16
            )

        RS1A, RS1B = 0, 2 * C
        RS2A, RS2B = 4 * C, 5 * C
        AG1A, AG1B = 6 * C, 7 * C
        AG2AQ, AG2AO = 8 * C, 9 * C
        AG2BQ, AG2BO = 10 * C, 11 * C

        a_rel = [pa_first + j * E for j in range(C)] + [
            Q - pa_first + j * E for j in range(C)
        ]
        b_rel = [pb_first + j * E for j in range(C)] + [
            Q - pb_first + j * E for j in range(C)
        ]
        rs1a, rs1b = [], []
        for j in range(2 * C):
            cast(sa + a_rel[j], E)
            r = copy(
                w.at[pl.ds(sa + a_rel[j], E)],
                rbuf1.at[0, pl.ds(a_rel[j], E)], RS1A + j, p1a,
            )
            r.start()
            rs1a.append(r)
            cast(sb + b_rel[j], E)
            r = copy(
                w.at[pl.ds(sb + b_rel[j], E)],
                rbuf1.at[1, pl.ds(b_rel[j], E)], RS1B + j, p1b,
            )
            r.start()
            rs1b.append(r)
        cast(ha, H)
        cast(hb, H)

        oq_rel = [(1 - ga) * Q, (1 - gb) * Q]
        q_rel = [ga * Q, gb * Q]
        rs2a, rs2b = [], []
        for j in range(C):
            rs1a[j].wait()
            w[pl.ds(oqa + j * E, E), :] = (
                w[pl.ds(oqa + j * E, E), :]
                + rbuf1[0, pl.ds(oq_rel[0] + j * E, E), :]
            )
            r = copy(w.at[pl.ds(oqa + j * E, E)], rbuf2.at[0, pl.ds(j * E, E)],
                     RS2A + j, p2a)
            r.start()
            rs2a.append(r)
            rs1b[j].wait()
            w[pl.ds(oqb + j * E, E), :] = (
                w[pl.ds(oqb + j * E, E), :]
                + rbuf1[1, pl.ds(oq_rel[1] + j * E, E), :]
            )
            r = copy(w.at[pl.ds(oqb + j * E, E)], rbuf2.at[1, pl.ds(j * E, E)],
                     RS2B + j, p2b)
            r.start()
            rs2b.append(r)
        for j in range(C):
            rs1a[C + j].wait()
            w[pl.ds(qa + j * E, E), :] = (
                w[pl.ds(qa + j * E, E), :]
                + rbuf1[0, pl.ds(q_rel[0] + j * E, E), :]
            )
            rs1b[C + j].wait()
            w[pl.ds(qb + j * E, E), :] = (
                w[pl.ds(qb + j * E, E), :]
                + rbuf1[1, pl.ds(q_rel[1] + j * E, E), :]
            )

        ag1a, ag1b, ag2 = [], [], []
        for j in range(C):
            rs2a[j].wait()
            w[pl.ds(qa + j * E, E), :] = (
                w[pl.ds(qa + j * E, E), :] + rbuf2[0, pl.ds(j * E, E), :]
            )
            r = copy(w.at[pl.ds(qa + j * E, E)], w.at[pl.ds(qa + j * E, E)],
                     AG1A + j, p2a)
            r.start()
            ag1a.append(r)
            r = copy(w.at[pl.ds(qa + j * E, E)], w.at[pl.ds(qa + j * E, E)],
                     AG2AQ + j, p1a)
            r.start()
            ag2.append(r)
            rs2b[j].wait()
            w[pl.ds(qb + j * E, E), :] = (
                w[pl.ds(qb + j * E, E), :] + rbuf2[1, pl.ds(j * E, E), :]
            )
            r = copy(w.at[pl.ds(qb + j * E, E)], w.at[pl.ds(qb + j * E, E)],
                     AG1B + j, p2b)
            r.start()
            ag1b.append(r)
            r = copy(w.at[pl.ds(qb + j * E, E)], w.at[pl.ds(qb + j * E, E)],
                     AG2BQ + j, p1b)
            r.start()
            ag2.append(r)

        for j in range(C):
            ag1a[j].wait()
            r = copy(w.at[pl.ds(oqa + j * E, E)], w.at[pl.ds(oqa + j * E, E)],
                     AG2AO + j, p1a)
            r.start()
            ag2.append(r)
            ag1b[j].wait()
            r = copy(w.at[pl.ds(oqb + j * E, E)], w.at[pl.ds(oqb + j * E, E)],
                     AG2BO + j, p1b)
            r.start()
            ag2.append(r)

        for r in ag2:
            r.wait()

    return pl.pallas_call(
        body,
        out_shape=jax.ShapeDtypeStruct((m, n), jnp.bfloat16),
        in_specs=[pl.BlockSpec(memory_space=pltpu.VMEM)],
        out_specs=pl.BlockSpec(memory_space=pltpu.VMEM),
        scratch_shapes=[
            pltpu.VMEM((2, H, n), jnp.bfloat16),
            pltpu.VMEM((2, Q, n), jnp.bfloat16),
            pltpu.SemaphoreType.DMA((NS,)),
            pltpu.SemaphoreType.DMA((NS,)),
        ],
        compiler_params=pltpu.CompilerParams(collective_id=0),
    )(x)
